# Initial kernel scaffold; baseline (speedup 1.0000x reference)
import jax
import jax.numpy as jnp
from jax.experimental import pallas as pl


def kernel(x, W_embed):
    raise NotImplementedError("write your pallas kernel here")


# speedup vs baseline: 1.7976x; 1.7976x over previous
import functools

import jax
import jax.numpy as jnp
from jax import lax
from jax.experimental import pallas as pl
from jax.experimental.pallas import tpu as pltpu
from jax.experimental.pallas import tpu_sc as plsc

VOCAB = 1000000
D = 64
IDX_W = 128
NUM_WORKERS = 32
K = 4


def _make_kernel(num_rows):
    rows_per_w = num_rows // NUM_WORKERS
    num_chunks = rows_per_w // K
    mesh = plsc.VectorSubcoreMesh(core_axis_name="c", subcore_axis_name="s")

    @functools.partial(
        pl.kernel,
        out_type=jax.ShapeDtypeStruct((num_rows, IDX_W, D), jnp.float32),
        mesh=mesh,
        scratch_types=[
            pltpu.VMEM((K, IDX_W), jnp.int32),
            pltpu.VMEM((K, IDX_W, D), jnp.float32),
            pltpu.SemaphoreType.DMA,
        ],
        compiler_params=pltpu.CompilerParams(use_tc_tiling_on_sc=False),
    )
    def emb(table_hbm, idx_hbm, out_hbm, idx_v, rows_v, sem):
        wid = lax.axis_index("s") * 2 + lax.axis_index("c")
        base_row = wid * rows_per_w

        @pl.loop(0, num_chunks)
        def chunk(c):
            row0 = base_row + c * K
            pltpu.sync_copy(idx_hbm.at[pl.ds(row0, K)], idx_v)
            copies = [
                pltpu.async_copy(table_hbm.at[idx_v.at[j]], rows_v.at[j], sem)
                for j in range(K)
            ]
            for cp in copies:
                cp.wait()
            pltpu.sync_copy(rows_v, out_hbm.at[pl.ds(row0, K)])

    return emb


def kernel(x, W_embed):
    b0, b1 = x.shape
    flat = x.reshape(-1).astype(jnp.int32)
    num_rows = flat.shape[0] // IDX_W
    idx2d = flat.reshape(num_rows, IDX_W)
    out = _make_kernel(num_rows)(W_embed, idx2d)
    return out.reshape(b0, b1, D)


# speedup vs baseline: 1.8701x; 1.0403x over previous
import functools

import jax
import jax.numpy as jnp
from jax import lax
from jax.experimental import pallas as pl
from jax.experimental.pallas import tpu as pltpu
from jax.experimental.pallas import tpu_sc as plsc

D = 64
IDX_W = 128
NUM_WORKERS = 32
K = 4


def _make_kernel(num_rows):
    rows_per_w = num_rows // NUM_WORKERS
    num_chunks = rows_per_w // K
    assert rows_per_w % K == 0 and num_chunks % 2 == 0 and num_chunks >= 6
    mesh = plsc.VectorSubcoreMesh(core_axis_name="c", subcore_axis_name="s")

    @functools.partial(
        pl.kernel,
        out_type=jax.ShapeDtypeStruct((num_rows, IDX_W, D), jnp.float32),
        mesh=mesh,
        scratch_types=[
            pltpu.VMEM((2, K, IDX_W), jnp.int32),
            pltpu.VMEM((2, K, IDX_W, D), jnp.float32),
            pltpu.SemaphoreType.DMA,
            pltpu.SemaphoreType.DMA,
            pltpu.SemaphoreType.DMA,
            pltpu.SemaphoreType.DMA,
            pltpu.SemaphoreType.DMA,
        ],
        compiler_params=pltpu.CompilerParams(use_tc_tiling_on_sc=False),
    )
    def emb(table_hbm, idx_hbm, out_hbm, idx_v, rows_v, gsem,
            isem0, isem1, osem0, osem1):
        wid = lax.axis_index("s") * 2 + lax.axis_index("c")
        base_row = wid * rows_per_w
        isem = (isem0, isem1)
        osem = (osem0, osem1)

        def idx_start(c, b):
            pltpu.async_copy(
                idx_hbm.at[pl.ds(base_row + c * K, K)], idx_v.at[b], isem[b])

        def idx_wait(c, b):
            pltpu.make_async_copy(
                idx_hbm.at[pl.ds(base_row + c * K, K)], idx_v.at[b],
                isem[b]).wait()

        def gather(b):
            copies = [
                pltpu.async_copy(table_hbm.at[idx_v.at[b, j]],
                                 rows_v.at[b, j], gsem)
                for j in range(K)
            ]
            for cp in copies:
                cp.wait()

        def out_start(c, b):
            pltpu.async_copy(
                rows_v.at[b], out_hbm.at[pl.ds(base_row + c * K, K)], osem[b])

        def out_wait(c, b):
            pltpu.make_async_copy(
                rows_v.at[b], out_hbm.at[pl.ds(base_row + c * K, K)],
                osem[b]).wait()

        idx_start(0, 0)
        idx_start(1, 1)
        for b in range(2):
            idx_wait(b, b)
            gather(b)
            out_start(b, b)
            idx_start(b + 2, b)

        @pl.loop(2, num_chunks - 2, step=2)
        def body(c0):
            for b in range(2):
                c = c0 + b
                idx_wait(c, b)
                out_wait(c - 2, b)
                gather(b)
                out_start(c, b)
                idx_start(c + 2, b)

        for b in range(2):
            c = num_chunks - 2 + b
            idx_wait(c, b)
            out_wait(c - 2, b)
            gather(b)
            out_start(c, b)
        for b in range(2):
            out_wait(num_chunks - 2 + b, b)

    return emb


def kernel(x, W_embed):
    b0, b1 = x.shape
    flat = x.reshape(-1).astype(jnp.int32)
    num_rows = flat.shape[0] // IDX_W
    idx2d = flat.reshape(num_rows, IDX_W)
    out = _make_kernel(num_rows)(W_embed, idx2d)
    return out.reshape(b0, b1, D)
